# baseline (speedup 1.0000x reference)
import functools
import math

import numpy as np
import jax
import jax.numpy as jnp
from jax import lax
from jax.experimental import pallas as pl
from jax.experimental.pallas import tpu as pltpu
from jax.experimental.pallas import tpu_sc as plsc

_B = 16
_D = 8
_T = 2048
_K = 4
_M = 3
_P = 6
_PP = _P * _P
_NSIG = _B * _D
_W = _T - _M + 1
_NTRANS = _W - 1
_NBINS = 40
_LANES = 16
_NREG = 8
_RBLK = _NBINS * _LANES
_HIST = _NREG * _RBLK
_F = _PP + _P


def _circulant_filters() -> np.ndarray:
    freqs = np.fft.fftfreq(_T)
    center = (np.arange(_K) - _K / 2.0) / _K
    bw = 1.0 / _K
    mask = np.exp(-0.5 * ((np.abs(freqs[None, :] - center[:, None])) / bw) ** 2)
    g = np.real(np.fft.ifft(mask, axis=1))
    idx = (np.arange(_T)[None, :] - np.arange(_T)[:, None]) % _T
    return np.ascontiguousarray(g[:, idx]).astype(np.float32)


_MFILT = _circulant_filters()
_JBLK = _T // 2


def _vmd_body(x_ref, m_ref, lin_ref, fvec_ref, modes_s):
    k = pl.program_id(0)
    j = pl.program_id(1)

    mcol = jnp.dot(x_ref[...], m_ref[0],
                   preferred_element_type=jnp.float32)
    modes_s[k, :, pl.ds(j * _JBLK, _JBLK)] = mcol

    @pl.when(j == 1)
    def _():
        modes = modes_s[k]
        m0 = modes[:, 0:_W]
        m1 = modes[:, 1:_W + 1]
        m2 = modes[:, 2:_W + 2]
        a = (m1 < m0).astype(jnp.int32)
        b = (m2 < m0).astype(jnp.int32)
        d = (m2 < m1).astype(jnp.int32)
        ids = 2 * a + b + d - a * d + 2 * b * d
        lin = ids[:, :_W - 1] * _P + ids[:, 1:]
        pad = jnp.full((_NSIG, _T - _NTRANS), _PP, jnp.int32)
        lin_ref[0] = jnp.concatenate([lin, pad], axis=-1)

    @pl.when(jnp.logical_and(k == _K - 1, j == 1))
    def _():
        mm = modes_s[...]
        e = mm * mm
        s1 = jnp.sum(e, axis=-1)
        n = float(_T)
        covd = []
        for i in range(_K):
            covd.append(jnp.sum(e[i] * e[i], axis=-1) - s1[i] * s1[i] / n)
        outs = []
        for i in range(_K):
            for jj in range(i + 1, _K):
                cij = jnp.sum(e[i] * e[jj], axis=-1) - s1[i] * s1[jj] / n
                den = jnp.sqrt(jnp.maximum(covd[i], 0.0)
                               * jnp.maximum(covd[jj], 0.0))
                outs.append(jnp.where(den > 0, cij / den, 0.0))
        fv = jnp.stack(outs, axis=-1)
        fvec_ref[...] = jnp.concatenate(
            [fv, jnp.zeros((_NSIG, 2), jnp.float32)], axis=-1)


def _vmd(x, mfilt):
    return pl.pallas_call(
        _vmd_body,
        grid=(_K, 2),
        in_specs=[
            pl.BlockSpec((_NSIG, _T), lambda k, j: (0, 0)),
            pl.BlockSpec((1, _T, _JBLK), lambda k, j: (k, 0, j)),
        ],
        out_specs=[
            pl.BlockSpec((1, _NSIG, _T), lambda k, j: (k, 0, 0)),
            pl.BlockSpec((_NSIG, 8), lambda k, j: (0, 0)),
        ],
        out_shape=[
            jax.ShapeDtypeStruct((_K, _NSIG, _T), jnp.int32),
            jax.ShapeDtypeStruct((_NSIG, 8), jnp.float32),
        ],
        scratch_shapes=[
            pltpu.VMEM((_K, _NSIG, _T), jnp.float32),
        ],
    )(x, mfilt)


def _sc_hist(lin, fvec):
    nc, ns = 2, 16
    nw = nc * ns
    spw = _NSIG // nw
    mesh = plsc.VectorSubcoreMesh(core_axis_name="c", subcore_axis_name="s")

    @functools.partial(
        pl.kernel,
        mesh=mesh,
        compiler_params=pltpu.CompilerParams(use_tc_tiling_on_sc=False,
                                             needs_layout_passes=False),
        out_type=jax.ShapeDtypeStruct((_NSIG * _F,), jnp.float32),
        scratch_types=[
            pltpu.VMEM((_K, spw, _T), jnp.int32),
            pltpu.VMEM((spw * _HIST,), jnp.float32),
            pltpu.VMEM((768,), jnp.float32),
            pltpu.VMEM((128,), jnp.float32),
            pltpu.VMEM((16,), jnp.float32),
            pltpu.VMEM((spw * 8 + 16,), jnp.float32),
            pltpu.VMEM((spw * _F + 32,), jnp.float32),
        ],
    )
    def run(lin_hbm, fvec_hbm, out_hbm, lin_v, hist_v, red_v, tot_v, rows_v,
            fvec_v, out_v):
        wid = lax.axis_index("s") * nc + lax.axis_index("c")
        base = wid * spw
        pltpu.sync_copy(lin_hbm.at[:, pl.ds(base, spw), :], lin_v)
        pltpu.sync_copy(fvec_hbm.at[pl.ds(base * 8, spw * 8)],
                        fvec_v.at[pl.ds(0, spw * 8)])
        lanes = lax.iota(jnp.int32, _LANES)
        ones = jnp.ones((_LANES,), jnp.float32)
        zeros = jnp.zeros((_LANES,), jnp.float32)

        def zbody(q):
            hist_v[pl.ds(q * _LANES, _LANES)] = zeros
        plsc.parallel_loop(0, spw * _HIST // _LANES, 1, unroll=4)(zbody)
        for q in range(640 // _LANES, 768 // _LANES):
            red_v[pl.ds(q * _LANES, _LANES)] = zeros
        for q in range(128 // _LANES):
            tot_v[pl.ds(q * _LANES, _LANES)] = zeros
        fvec_v[pl.ds(spw * 8, _LANES)] = zeros

        for s in range(spw):
            for k in range(_K):
                def body(t):
                    v = lin_v[k, s, pl.ds(t * _LANES, _LANES)]
                    reg = lax.rem(t, _NREG)
                    off = s * _HIST + reg * _RBLK
                    idx = v * _LANES + lanes + off
                    plsc.addupdate_scatter(hist_v, [idx], ones)
                plsc.parallel_loop(0, _T // _LANES, 1, unroll=_NREG)(body)

        def _rowidx(first_bin):
            bb = lanes + first_bin
            r = jnp.zeros((_LANES,), jnp.int32)
            for th in range(6, _PP, 6):
                r = r + (bb >= th).astype(jnp.int32)
            return jnp.minimum(r, 5)
        rowidx = [_rowidx(0), _rowidx(16), _rowidx(32)]

        for s in range(spw):
            for b in range(_NBINS):
                o = s * _HIST + b * _LANES
                v = hist_v[pl.ds(o, _LANES)]
                for r in range(1, _NREG):
                    v = v + hist_v[pl.ds(o + r * _RBLK, _LANES)]
                red_v[pl.ds(b * _LANES, _LANES)] = v
            tots = []
            for c in range(3):
                acc = zeros
                for l in range(_LANES):
                    gidx = lanes * _LANES + (c * 256 + l)
                    acc = acc + plsc.load_gather(red_v, [gidx])
                tots.append(acc)
                tot_v[pl.ds(c * _LANES, _LANES)] = acc
            rows = zeros
            for c in range(_P):
                rows = rows + plsc.load_gather(tot_v, [lanes * _P + c])
            rows_v[...] = jnp.where(rows == 0.0, 1.0, rows)
            for c in range(3):
                rs = plsc.load_gather(rows_v, [rowidx[c]])
                norm = tots[c] / rs
                if c < 2:
                    out_v[pl.ds(s * _F + c * _LANES, _LANES)] = norm
                else:
                    plsc.store_compressed(
                        out_v.at[pl.ds(s * _F + 32, _LANES)], norm,
                        mask=(lanes < 4))
            fv = fvec_v[pl.ds(s * 8, _LANES)]
            plsc.store_compressed(
                out_v.at[pl.ds(s * _F + _PP, _LANES)], fv, mask=(lanes < _P))

        pltpu.sync_copy(out_v.at[pl.ds(0, spw * _F)],
                        out_hbm.at[pl.ds(base * _F, spw * _F)])

    return run(lin, fvec)


def kernel(hidden_states):
    x = hidden_states.transpose(0, 2, 1).reshape(_NSIG, _T)
    lin, fvec = _vmd(x, jnp.asarray(_MFILT))
    out = _sc_hist(lin, fvec.reshape(_NSIG * 8))
    return out.reshape(_B, _D * _F)


# speedup vs baseline: 1.0609x; 1.0609x over previous
import functools
import math

import numpy as np
import jax
import jax.numpy as jnp
from jax import lax
from jax.experimental import pallas as pl
from jax.experimental.pallas import tpu as pltpu
from jax.experimental.pallas import tpu_sc as plsc

_B = 16
_D = 8
_T = 2048
_K = 4
_M = 3
_P = 6
_PP = _P * _P
_NSIG = _B * _D
_W = _T - _M + 1
_NTRANS = _W - 1
_NBINS = 40
_LANES = 16
_NREG = 4
_RBLK = _NBINS * _LANES
_HIST = _NREG * _RBLK
_F = _PP + _P


def _circulant_filters() -> np.ndarray:
    freqs = np.fft.fftfreq(_T)
    center = (np.arange(_K) - _K / 2.0) / _K
    bw = 1.0 / _K
    mask = np.exp(-0.5 * ((np.abs(freqs[None, :] - center[:, None])) / bw) ** 2)
    g = np.real(np.fft.ifft(mask, axis=1))
    idx = (np.arange(_T)[None, :] - np.arange(_T)[:, None]) % _T
    return np.ascontiguousarray(g[:, idx]).astype(np.float32)


_MFILT = _circulant_filters()
_JBLK = _T // 2


def _vmd_body(x_ref, m_ref, lin_ref, fvec_ref, modes_s):
    k = pl.program_id(0)
    j = pl.program_id(1)

    mcol = jnp.dot(x_ref[...], m_ref[0],
                   preferred_element_type=jnp.float32)
    modes_s[k, :, pl.ds(j * _JBLK, _JBLK)] = mcol

    @pl.when(j == 1)
    def _():
        modes = modes_s[k]
        m0 = modes[:, 0:_W]
        m1 = modes[:, 1:_W + 1]
        m2 = modes[:, 2:_W + 2]
        a = (m1 < m0).astype(jnp.int32)
        b = (m2 < m0).astype(jnp.int32)
        d = (m2 < m1).astype(jnp.int32)
        ids = 2 * a + b + d - a * d + 2 * b * d
        lin = ids[:, :_W - 1] * _P + ids[:, 1:]
        pad = jnp.full((_NSIG, _T - _NTRANS), _PP, jnp.int32)
        lin_ref[0] = jnp.concatenate([lin, pad], axis=-1)

    @pl.when(jnp.logical_and(k == _K - 1, j == 1))
    def _():
        mm = modes_s[...]
        e = mm * mm
        s1 = jnp.sum(e, axis=-1)
        n = float(_T)
        covd = []
        for i in range(_K):
            covd.append(jnp.sum(e[i] * e[i], axis=-1) - s1[i] * s1[i] / n)
        outs = []
        for i in range(_K):
            for jj in range(i + 1, _K):
                cij = jnp.sum(e[i] * e[jj], axis=-1) - s1[i] * s1[jj] / n
                den = jnp.sqrt(jnp.maximum(covd[i], 0.0)
                               * jnp.maximum(covd[jj], 0.0))
                outs.append(jnp.where(den > 0, cij / den, 0.0))
        fv = jnp.stack(outs, axis=-1)
        fvec_ref[...] = jnp.concatenate(
            [fv, jnp.zeros((_NSIG, 2), jnp.float32)], axis=-1)


def _vmd(x, mfilt):
    return pl.pallas_call(
        _vmd_body,
        grid=(_K, 2),
        in_specs=[
            pl.BlockSpec((_NSIG, _T), lambda k, j: (0, 0)),
            pl.BlockSpec((1, _T, _JBLK), lambda k, j: (k, 0, j)),
        ],
        out_specs=[
            pl.BlockSpec((1, _NSIG, _T), lambda k, j: (k, 0, 0)),
            pl.BlockSpec((_NSIG, 8), lambda k, j: (0, 0)),
        ],
        out_shape=[
            jax.ShapeDtypeStruct((_K, _NSIG, _T), jnp.int32),
            jax.ShapeDtypeStruct((_NSIG, 8), jnp.float32),
        ],
        scratch_shapes=[
            pltpu.VMEM((_K, _NSIG, _T), jnp.float32),
        ],
    )(x, mfilt)


def _sc_hist(lin, fvec):
    nc, ns = 2, 16
    nw = nc * ns
    spw = _NSIG // nw
    mesh = plsc.VectorSubcoreMesh(core_axis_name="c", subcore_axis_name="s")

    @functools.partial(
        pl.kernel,
        mesh=mesh,
        compiler_params=pltpu.CompilerParams(use_tc_tiling_on_sc=False,
                                             needs_layout_passes=False),
        out_type=jax.ShapeDtypeStruct((_NSIG * _F,), jnp.float32),
        scratch_types=[
            pltpu.VMEM((_K, spw, _T), jnp.int32),
            pltpu.VMEM((spw * _HIST,), jnp.float32),
            pltpu.VMEM((768,), jnp.float32),
            pltpu.VMEM((128,), jnp.float32),
            pltpu.VMEM((16,), jnp.float32),
            pltpu.VMEM((spw * 8 + 16,), jnp.float32),
            pltpu.VMEM((spw * _F + 32,), jnp.float32),
        ],
    )
    def run(lin_hbm, fvec_hbm, out_hbm, lin_v, hist_v, red_v, tot_v, rows_v,
            fvec_v, out_v):
        wid = lax.axis_index("s") * nc + lax.axis_index("c")
        base = wid * spw
        pltpu.sync_copy(lin_hbm.at[:, pl.ds(base, spw), :], lin_v)
        pltpu.sync_copy(fvec_hbm.at[pl.ds(base * 8, spw * 8)],
                        fvec_v.at[pl.ds(0, spw * 8)])
        lanes = lax.iota(jnp.int32, _LANES)
        ones = jnp.ones((_LANES,), jnp.float32)
        zeros = jnp.zeros((_LANES,), jnp.float32)

        def zbody(q):
            hist_v[pl.ds(q * _LANES, _LANES)] = zeros
        plsc.parallel_loop(0, spw * _HIST // _LANES, 1, unroll=4)(zbody)
        for q in range(640 // _LANES, 768 // _LANES):
            red_v[pl.ds(q * _LANES, _LANES)] = zeros
        for q in range(128 // _LANES):
            tot_v[pl.ds(q * _LANES, _LANES)] = zeros
        fvec_v[pl.ds(spw * 8, _LANES)] = zeros

        for s in range(spw):
            for k in range(_K):
                def body(t):
                    v = lin_v[k, s, pl.ds(t * _LANES, _LANES)]
                    reg = lax.rem(t, _NREG)
                    off = s * _HIST + reg * _RBLK
                    idx = v * _LANES + lanes + off
                    plsc.addupdate_scatter(hist_v, [idx], ones)
                plsc.parallel_loop(0, _T // _LANES, 1, unroll=_NREG)(body)

        def _rowidx(first_bin):
            bb = lanes + first_bin
            r = jnp.zeros((_LANES,), jnp.int32)
            for th in range(6, _PP, 6):
                r = r + (bb >= th).astype(jnp.int32)
            return jnp.minimum(r, 5)
        rowidx = [_rowidx(0), _rowidx(16), _rowidx(32)]

        for s in range(spw):
            for b in range(_NBINS):
                o = s * _HIST + b * _LANES
                v = hist_v[pl.ds(o, _LANES)]
                for r in range(1, _NREG):
                    v = v + hist_v[pl.ds(o + r * _RBLK, _LANES)]
                red_v[pl.ds(b * _LANES, _LANES)] = v
            tots = []
            for c in range(3):
                acc = zeros
                for l in range(_LANES):
                    gidx = lanes * _LANES + (c * 256 + l)
                    acc = acc + plsc.load_gather(red_v, [gidx])
                tots.append(acc)
                tot_v[pl.ds(c * _LANES, _LANES)] = acc
            rows = zeros
            for c in range(_P):
                rows = rows + plsc.load_gather(tot_v, [lanes * _P + c])
            rows_v[...] = jnp.where(rows == 0.0, 1.0, rows)
            for c in range(3):
                rs = plsc.load_gather(rows_v, [rowidx[c]])
                norm = tots[c] / rs
                if c < 2:
                    out_v[pl.ds(s * _F + c * _LANES, _LANES)] = norm
                else:
                    plsc.store_compressed(
                        out_v.at[pl.ds(s * _F + 32, _LANES)], norm,
                        mask=(lanes < 4))
            fv = fvec_v[pl.ds(s * 8, _LANES)]
            plsc.store_compressed(
                out_v.at[pl.ds(s * _F + _PP, _LANES)], fv, mask=(lanes < _P))

        pltpu.sync_copy(out_v.at[pl.ds(0, spw * _F)],
                        out_hbm.at[pl.ds(base * _F, spw * _F)])

    return run(lin, fvec)


def kernel(hidden_states):
    x = hidden_states.transpose(0, 2, 1).reshape(_NSIG, _T)
    lin, fvec = _vmd(x, jnp.asarray(_MFILT))
    out = _sc_hist(lin, fvec.reshape(_NSIG * 8))
    return out.reshape(_B, _D * _F)
